# baseline (speedup 1.0000x reference)
import functools

import jax
import jax.numpy as jnp
import numpy as np
from jax.experimental import pallas as pl
from jax.experimental.pallas import tpu as pltpu

NB = 256
L = NB * 8
L2 = L - 2
CH = 32
SC = 5 * CH

_ROWS = dict(g0=(0, 384), g1=(384, 384), h0=(768, 384), h1=(1152, 384),
             v0=(1536, 256), v1=(1792, 256), v2=(2048, 256),
             f01=(2304, 256), f23=(2560, 256), f4=(2816, 128),
             wf2=(2944, 128), wf3=(3072, 128))
_WROWS = 3200


NSPLIT = 4


def _lenet_body(x0_ref, x1_ref, x2_ref, x3_ref, wc_hbm, s_hbm, bias_hbm,
                o_ref, wc, s, bias, sems):
    f32, bf16 = jnp.float32, jnp.bfloat16
    dot = functools.partial(jnp.dot, preferred_element_type=f32)

    @pl.when(pl.program_id(0) == 0)
    def _load_weights():
        pltpu.make_async_copy(wc_hbm, wc, sems.at[0]).start()
        pltpu.make_async_copy(s_hbm, s, sems.at[1]).start()
        pltpu.make_async_copy(bias_hbm, bias, sems.at[2]).start()
        pltpu.make_async_copy(wc_hbm, wc, sems.at[0]).wait()
        pltpu.make_async_copy(s_hbm, s, sems.at[1]).wait()
        pltpu.make_async_copy(bias_hbm, bias, sems.at[2]).wait()

    def W(name, lanes=512):
        r0, nr = _ROWS[name]
        return wc[r0:r0 + nr, :lanes]

    NC = NB // NSPLIT
    X = jnp.concatenate(
        [jnp.concatenate(
            [xr[0][:, c].reshape(NC * 8, 128).astype(bf16) for c in range(3)],
            axis=1)
         for xr in (x0_ref, x1_ref, x2_ref, x3_ref)], axis=0)
    X = jnp.concatenate([X, jnp.zeros((8, 384), bf16)], axis=0)

    def pool_relu(y, brow):
        m = jnp.maximum(y[:, :256], y[:, 256:])
        m = jnp.maximum(m[:, :128], m[:, 128:]) + bias[brow:brow + 1]
        return jnp.maximum(m, 0.0).astype(bf16)

    y01 = dot(X[0:L], W("g0")) + dot(X[1:1 + L], W("g1"))
    ze = pool_relu(y01, 0)
    y23 = dot(X[0:L], W("h0")) + dot(X[1:1 + L], W("h1"))
    zo = pool_relu(y23, 0)
    C = jnp.concatenate([ze, zo], axis=1)

    u = (dot(C[0:L2], W("v0")) + dot(C[1:1 + L2], W("v1"))
         + dot(C[2:2 + L2], W("v2")))
    p2 = pool_relu(u, 1)

    svals = s[...]
    h1s = []
    for c in range(NB // CH):
        p2c = p2[c * CH * 8:c * CH * 8 + SC * 2 - 66]
        sg = dot(svals, p2c).astype(bf16)
        h1 = (dot(jnp.concatenate([sg[0:CH], sg[CH:2 * CH]], axis=1),
                  W("f01", 128))
              + dot(jnp.concatenate([sg[2 * CH:3 * CH], sg[3 * CH:4 * CH]],
                                    axis=1), W("f23", 128))
              + dot(sg[4 * CH:5 * CH], W("f4", 128)))
        h1s.append(jnp.maximum(h1 + bias[2:3], 0.0).astype(bf16))
    h1 = jnp.concatenate(h1s, axis=0)
    h2 = jnp.maximum(dot(h1, W("wf2", 128)) + bias[3:4], 0.0).astype(bf16)
    o_ref[0] = dot(h2, W("wf3", 128)) + bias[4:5]


def _pack_weights(wb1, wb2, wf1):
    bf16 = jnp.bfloat16
    z = jnp.zeros((128, 256), bf16)

    def pair(wa, wb, wc_, wd):
        return jnp.concatenate(
            [jnp.concatenate([wa, wb], axis=0),
             jnp.concatenate([wc_, wd], axis=0)], axis=1)

    w0 = pair(wb1[0], wb1[1], z, wb1[0])
    w1 = pair(wb1[2], wb1[3], wb1[1], wb1[2])
    w2 = pair(wb1[4], z, wb1[3], wb1[4])
    c, r4, w = np.meshgrid(np.arange(3), np.arange(4), np.arange(32),
                           indexing="ij")
    q = jnp.asarray((r4 * 128 + w * 4 + c).reshape(384))
    z512 = jnp.zeros((256, 512), bf16)
    vs = lambda a, b: jnp.concatenate([a, b], axis=0)
    parts = {
        "g0": vs(w0, w1)[q], "g1": vs(w2, z512)[q],
        "h0": vs(z512, w0)[q], "h1": vs(w1, w2)[q],
        "v0": pair(wb2[0], wb2[1], z, wb2[0]),
        "v1": pair(wb2[2], wb2[3], wb2[1], wb2[2]),
        "v2": pair(wb2[4], z, wb2[3], wb2[4]),
        "f01": jnp.concatenate([wf1[0], wf1[1]], axis=0),
        "f23": jnp.concatenate([wf1[2], wf1[3]], axis=0),
        "f4": wf1[4],
    }
    wc = jnp.zeros((_WROWS, 512), bf16)
    for name, (r0, nr) in _ROWS.items():
        if name in parts:
            p = parts[name]
            wc = wc.at[r0:r0 + nr, :p.shape[1]].set(p)
    g = np.arange(SC)
    cols = 8 * (g % CH) + g // CH
    s = np.zeros((SC, SC * 2 - 66), np.float32)
    s[g, cols] = 1.0
    return wc, jnp.asarray(s, bf16)


def kernel(x, wb1, b1, wb2, b2, wf1, bf1, wf2, bf2, wf3, bf3):
    Bt, Cc, H, Wd = x.shape
    assert (Cc, H, Wd) == (3, 32, 32)
    nsteps = max(1, -(-Bt // NB))
    Bp = nsteps * NB
    if Bp != Bt:
        x = jnp.pad(x, ((0, Bp - Bt), (0, 0), (0, 0), (0, 0)))
    xv = x.reshape(nsteps * NSPLIT, NB // NSPLIT, 3, 8, 128)

    wc, s = _pack_weights(wb1, wb2, wf1)
    wc = wc.at[_ROWS["wf2"][0]:_ROWS["wf2"][0] + 128, :128].set(wf2)
    wc = wc.at[_ROWS["wf3"][0]:_ROWS["wf3"][0] + 128, :128].set(wf3)
    bias = jnp.concatenate([b1, b2, bf1, bf2, bf3], axis=0)

    out = pl.pallas_call(
        _lenet_body,
        out_shape=jax.ShapeDtypeStruct((nsteps, NB, 128), jnp.float32),
        grid=(nsteps,),
        in_specs=[
            pl.BlockSpec((1, NB // NSPLIT, 3, 8, 128),
                         lambda i, j=j: (NSPLIT * i + j, 0, 0, 0, 0))
            for j in range(NSPLIT)
        ] + [
            pl.BlockSpec(memory_space=pl.ANY),
            pl.BlockSpec(memory_space=pl.ANY),
            pl.BlockSpec(memory_space=pl.ANY),
        ],
        out_specs=pl.BlockSpec((1, NB, 128), lambda i: (i, 0, 0)),
        scratch_shapes=[
            pltpu.VMEM((_WROWS, 512), jnp.bfloat16),
            pltpu.VMEM((SC, SC * 2 - 66), jnp.bfloat16),
            pltpu.VMEM((5, 128), jnp.float32),
            pltpu.SemaphoreType.DMA((3,)),
        ],
        compiler_params=pltpu.CompilerParams(
            dimension_semantics=("arbitrary",),
            vmem_limit_bytes=64 * 1024 * 1024),
    )(xv, xv, xv, xv, wc, s, bias)
    return out.reshape(Bp, 128)[:Bt, :10]
